# Initial kernel scaffold; baseline (speedup 1.0000x reference)
import jax
import jax.numpy as jnp
from jax.experimental import pallas as pl


def kernel(token_ids, cu_seqlens, table):
    raise NotImplementedError("write your pallas kernel here")


# speedup vs baseline: 1.3308x; 1.3308x over previous
import functools

import jax
import jax.numpy as jnp
from jax import lax
from jax.experimental import pallas as pl
from jax.experimental.pallas import tpu as pltpu
from jax.experimental.pallas import tpu_sc as plsc

_VOCAB = 32000
_D = 512
_T = 8192
_NC = 2
_NS = 16
_NW = _NC * _NS
_TPW = _T // _NW
_CHUNK = 128
_NCHUNK = _TPW // _CHUNK
_LANES = 16


def _bcast(vec, lane):
    idx = jnp.full((_LANES, 1), lane, jnp.int32)
    dnums = lax.GatherDimensionNumbers(
        offset_dims=(), collapsed_slice_dims=(0,), start_index_map=(0,))
    return lax.gather(vec, idx, dnums, (1,),
                      mode=lax.GatherScatterMode.PROMISE_IN_BOUNDS)


def _body(ids_hbm, cu_hbm, table_hbm, out_hbm, idx_v, rows_v, w_v, cu_v,
          acc_v, sem):
    cid = lax.axis_index("c")
    sid = lax.axis_index("s")
    wid = cid * _NS + sid
    base = wid * _TPW

    pltpu.sync_copy(cu_hbm, cu_v)
    pltpu.sync_copy(ids_hbm.at[pl.ds(base, _TPW)], idx_v)

    lanes = lax.iota(jnp.int32, 16)
    cuv = cu_v[...]
    cks = [_bcast(cuv, k) for k in range(9)]

    for jj in range(_TPW // _LANES):
        t = base + jj * _LANES + lanes
        start = jnp.zeros((_LANES,), jnp.int32)
        end = jnp.full((_LANES,), _T, jnp.int32)
        for ck in cks:
            start = jnp.maximum(start, jnp.where(ck <= t, ck, 0))
            end = jnp.minimum(end, jnp.where(ck > t, ck, _T))
        pos = t - start
        seg_len = end - start
        w = ((pos + 1) * (seg_len - pos)).astype(jnp.float32)
        w_v[pl.ds(jj * _LANES, _LANES)] = w

    acc = jnp.zeros((_LANES,), jnp.float32)
    for c in range(_NCHUNK):
        pltpu.async_copy(
            table_hbm.at[idx_v.at[pl.ds(c * _CHUNK, _CHUNK)]], rows_v, sem
        ).wait()

        def grp(g, acc, c=c):
            w16 = w_v[pl.ds(c * _CHUNK + g * _LANES, _LANES)]
            for r in range(_LANES):
                row = g * _LANES + r
                parts = [rows_v[row, pl.ds(j * _LANES, _LANES)]
                         for j in range(_D // _LANES)]
                while len(parts) > 1:
                    parts = [parts[i] + parts[i + 1]
                             for i in range(0, len(parts) - 1, 2)] + (
                                 [parts[-1]] if len(parts) % 2 else [])
                wr = _bcast(w16, r)
                acc = acc + wr * parts[0]
            return acc

        acc = lax.fori_loop(0, _CHUNK // _LANES, grp, acc)

    acc_v[...] = acc
    pltpu.sync_copy(acc_v, out_hbm.at[wid])


@functools.partial(jax.jit, static_argnames=())
def _run(ids, cu16, table):
    mesh = plsc.VectorSubcoreMesh(core_axis_name="c", subcore_axis_name="s")
    kern = pl.kernel(
        _body,
        out_type=jax.ShapeDtypeStruct((_NW, _LANES), jnp.float32),
        mesh=mesh,
        scratch_types=[
            pltpu.VMEM((_TPW,), jnp.int32),
            pltpu.VMEM((_CHUNK, _D), jnp.float32),
            pltpu.VMEM((_TPW,), jnp.float32),
            pltpu.VMEM((_LANES,), jnp.int32),
            pltpu.VMEM((_LANES,), jnp.float32),
            pltpu.SemaphoreType.DMA,
        ],
    )
    return kern(ids, cu16, table)


def kernel(token_ids, cu_seqlens, table):
    ids = token_ids.astype(jnp.int32)
    cu = cu_seqlens.astype(jnp.int32)
    pad = jnp.full((_LANES - cu.shape[0],), _T, jnp.int32)
    cu16 = jnp.concatenate([cu, pad])
    partials = _run(ids, cu16, table)
    return jnp.sum(partials)
